# baseline (speedup 1.0000x reference)
import functools

import jax
import jax.numpy as jnp
from jax import lax
from jax.experimental import pallas as pl
from jax.experimental.pallas import tpu as pltpu
from jax.experimental.pallas import tpu_sc as plsc

_B, _N = 1024, 200
_S, _V, _D = 4, 100000, 64
_LANES = 16
_R = _B * _S

_NUM_WORKERS = 32
_BAND = _R // _NUM_WORKERS
_GBUF = 4
_TBUF = 2
_DT = _D // 8
_TILE = _D * _BAND


def _sc_body(idx_hbm, tok_hbm, pos_hbm, out_hbm,
             idx_v, pos_v, stage_v, tile_v, gsem, ssem):
    wid = lax.axis_index("subcore") * 2 + lax.axis_index("core")
    col0 = pl.multiple_of(wid * _BAND, 8)

    pltpu.sync_copy(idx_hbm.at[:, pl.ds(col0, _BAND)], idx_v)
    pltpu.sync_copy(pos_hbm, pos_v)

    iota = lax.iota(jnp.int32, _LANES)
    d_base = [(iota + db * _LANES) * _BAND for db in range(_D // _LANES)]

    def gather_desc(n, slot):
        return pltpu.make_async_copy(
            tok_hbm.at[idx_v.at[n]],
            stage_v.at[pl.ds(slot * _BAND, _BAND)], gsem.at[slot])

    def store_descs(n, tb):
        return [pltpu.make_async_copy(
            tile_v.at[pl.ds(tb * _TILE + dt * (8 * _BAND), 8 * _BAND)],
            out_hbm.at[n, dt, wid], ssem.at[tb]) for dt in range(_DT)]

    for n in range(_GBUF - 1):
        gather_desc(n, n).start()

    def step(n2, carry):
        for par in range(_GBUF):
            n = n2 * _GBUF + par
            tb = par % _TBUF
            gather_desc(n, par).wait()

            @pl.when(n >= _TBUF)
            def _():
                for c in store_descs(n, tb):
                    c.wait()

            posv = [[pos_v[pl.ds(sv * (_N * _D) + n * _D + db * _LANES,
                                 _LANES)]
                     for db in range(_D // _LANES)] for sv in range(_S)]

            @plsc.parallel_loop(0, _BAND // _S, unroll=4)
            def row4(q):
                r0 = q * _S
                for sv in range(_S):
                    r = par * _BAND + r0 + sv
                    off = tb * _TILE + r0 + sv
                    for db in range(_D // _LANES):
                        vals = (stage_v[r, pl.ds(db * _LANES, _LANES)]
                                + posv[sv][db])
                        plsc.store_scatter(tile_v, [d_base[db] + off], vals)

            for c in store_descs(n, tb):
                c.start()

            @pl.when(n + _GBUF - 1 < _N)
            def _():
                gather_desc(n + _GBUF - 1, (par + _GBUF - 1) % _GBUF).start()
        return carry

    lax.fori_loop(0, _N // _GBUF, step, 0)

    for c in store_descs(_N - 2, (_N - 2) % _TBUF):
        c.wait()
    for c in store_descs(_N - 1, (_N - 1) % _TBUF):
        c.wait()


_sc_call = functools.partial(
    pl.kernel,
    out_type=jax.ShapeDtypeStruct((_N, _DT, _NUM_WORKERS, 8 * _BAND),
                                  jnp.float32),
    mesh=plsc.VectorSubcoreMesh(core_axis_name="core",
                                subcore_axis_name="subcore"),
    scratch_types=[
        pltpu.VMEM((_N, _BAND), jnp.int32),
        pltpu.VMEM((_S * _N * _D,), jnp.float32),
        pltpu.VMEM((_GBUF * _BAND, _D), jnp.float32),
        pltpu.VMEM((_TBUF * _TILE,), jnp.float32),
        pltpu.SemaphoreType.DMA((_GBUF,)),
        pltpu.SemaphoreType.DMA((_TBUF,)),
    ],
    compiler_params=pltpu.CompilerParams(use_tc_tiling_on_sc=False,
                                         needs_layout_passes=False),
)(_sc_body)


def kernel(x, token_emb, pos_emb):
    tok_flat = token_emb.reshape(_S * _V, _D)
    offs = jnp.arange(_S, dtype=jnp.int32) * _V
    idx_t = (x.T.astype(jnp.int32)[:, :, None]
             + offs[None, None, :]).reshape(_N, _R)
    pos_flat = pos_emb[:, :_N, :].reshape(_S * _N * _D)
    out_t = _sc_call(idx_t, tok_flat, pos_flat)
    out_t = out_t.reshape(_N, _DT, _NUM_WORKERS, 8, _BAND)
    return out_t.transpose(2, 4, 0, 1, 3).reshape(_R, _N, _D)


# speedup vs baseline: 1.8946x; 1.8946x over previous
import functools

import jax
import jax.numpy as jnp
from jax import lax
from jax.experimental import pallas as pl
from jax.experimental.pallas import tpu as pltpu
from jax.experimental.pallas import tpu_sc as plsc

_B, _N = 1024, 200
_S, _V, _D = 4, 100000, 64
_LANES = 16
_R = _B * _S

_NUM_WORKERS = 32
_BAND = _R // _NUM_WORKERS
_GBUF = 4
_TBUF = 2
_DT = _D // 8
_TILE = _D * _BAND


def _sc_body(idx_hbm, tok_hbm, pos_hbm, out_hbm,
             idx_v, pos_v, stage_v, tile_v, gsem, ssem):
    wid = lax.axis_index("subcore") * 2 + lax.axis_index("core")
    col0 = pl.multiple_of(wid * _BAND, 8)

    pltpu.sync_copy(idx_hbm.at[:, pl.ds(col0, _BAND)], idx_v)
    pltpu.sync_copy(pos_hbm, pos_v)

    iota = lax.iota(jnp.int32, _LANES)
    d_vecs = [iota + db * _LANES for db in range(_D // _LANES)]

    def gather_desc(n, slot):
        return pltpu.make_async_copy(
            tok_hbm.at[idx_v.at[n]],
            stage_v.at[pl.ds(slot * _BAND, _BAND)], gsem.at[slot])

    def store_descs(n, tb):
        return [pltpu.make_async_copy(
            tile_v.at[tb, pl.ds(dt * 8, 8), pl.ds(0, _BAND)],
            out_hbm.at[n, dt, wid], ssem.at[tb]) for dt in range(_DT)]

    for n in range(_GBUF - 1):
        gather_desc(n, n).start()

    def step(n2, carry):
        for par in range(_GBUF):
            n = n2 * _GBUF + par
            tb = par % _TBUF
            gather_desc(n, par).wait()

            @pl.when(n >= _TBUF)
            def _():
                for c in store_descs(n, tb):
                    c.wait()

            posv = [[pos_v[pl.ds(sv * (_N * _D) + n * _D + db * _LANES,
                                 _LANES)]
                     for db in range(_D // _LANES)] for sv in range(_S)]
            tb_spl = jnp.full((_LANES,), tb, jnp.int32)

            @plsc.parallel_loop(0, _BAND // _S, unroll=4)
            def row4(q):
                r0 = q * _S
                for sv in range(_S):
                    r = par * _BAND + r0 + sv
                    r_spl = jnp.full((_LANES,), r0 + sv, jnp.int32)
                    for db in range(_D // _LANES):
                        vals = (stage_v[r, pl.ds(db * _LANES, _LANES)]
                                + posv[sv][db])
                        plsc.store_scatter(
                            tile_v, [tb_spl, d_vecs[db], r_spl], vals)

            for c in store_descs(n, tb):
                c.start()

            @pl.when(n + _GBUF - 1 < _N)
            def _():
                gather_desc(n + _GBUF - 1, (par + _GBUF - 1) % _GBUF).start()
        return carry

    lax.fori_loop(0, _N // _GBUF, step, 0)

    for c in store_descs(_N - 2, (_N - 2) % _TBUF):
        c.wait()
    for c in store_descs(_N - 1, (_N - 1) % _TBUF):
        c.wait()


_sc_call = functools.partial(
    pl.kernel,
    out_type=jax.ShapeDtypeStruct((_N, _DT, _NUM_WORKERS, 8, _BAND),
                                  jnp.float32),
    mesh=plsc.VectorSubcoreMesh(core_axis_name="core",
                                subcore_axis_name="subcore"),
    scratch_types=[
        pltpu.VMEM((_N, _BAND), jnp.int32),
        pltpu.VMEM((_S * _N * _D,), jnp.float32),
        pltpu.VMEM((_GBUF * _BAND, _D), jnp.float32),
        pltpu.VMEM((_TBUF, _D, _BAND + 1), jnp.float32),
        pltpu.SemaphoreType.DMA((_GBUF,)),
        pltpu.SemaphoreType.DMA((_TBUF,)),
    ],
    compiler_params=pltpu.CompilerParams(use_tc_tiling_on_sc=False,
                                         needs_layout_passes=False),
)(_sc_body)


def kernel(x, token_emb, pos_emb):
    tok_flat = token_emb.reshape(_S * _V, _D)
    offs = jnp.arange(_S, dtype=jnp.int32) * _V
    idx_t = (x.T.astype(jnp.int32)[:, :, None]
             + offs[None, None, :]).reshape(_N, _R)
    pos_flat = pos_emb[:, :_N, :].reshape(_S * _N * _D)
    out_t = _sc_call(idx_t, tok_flat, pos_flat)
    return out_t.transpose(2, 4, 0, 1, 3).reshape(_R, _N, _D)


# speedup vs baseline: 1.9899x; 1.0503x over previous
import functools

import jax
import jax.numpy as jnp
from jax import lax
from jax.experimental import pallas as pl
from jax.experimental.pallas import tpu as pltpu
from jax.experimental.pallas import tpu_sc as plsc

_B, _N = 1024, 200
_S, _V, _D = 4, 100000, 64
_LANES = 16
_R = _B * _S

_NUM_WORKERS = 32
_BAND = _R // _NUM_WORKERS
_GBUF = 4
_TBUF = 2
_DT = _D // 8
_TILE = _D * _BAND


def _sc_body(idx_hbm, tok_hbm, pos_hbm, out_hbm,
             idx_v, pos_v, stage_v, tile_v, gsem, ssem):
    wid = lax.axis_index("subcore") * 2 + lax.axis_index("core")
    col0 = pl.multiple_of(wid * _BAND, 8)

    pltpu.sync_copy(idx_hbm.at[:, pl.ds(col0, _BAND)], idx_v)
    pltpu.sync_copy(pos_hbm, pos_v)

    iota = lax.iota(jnp.int32, _LANES)
    d_vecs = [iota + db * _LANES for db in range(_D // _LANES)]

    def gather_desc(n, slot):
        return pltpu.make_async_copy(
            tok_hbm.at[idx_v.at[n]],
            stage_v.at[pl.ds(slot * _BAND, _BAND)], gsem.at[slot])

    def store_descs(n, tb):
        return [pltpu.make_async_copy(
            tile_v.at[tb, pl.ds(dt * 8, 8), pl.ds(0, _BAND)],
            out_hbm.at[n, dt, wid], ssem.at[tb]) for dt in range(_DT)]

    for n in range(_GBUF - 1):
        gather_desc(n, n).start()

    def step(n2, carry):
        for par in range(_GBUF):
            n = n2 * _GBUF + par
            tb = par % _TBUF
            gather_desc(n, par).wait()

            @pl.when(n >= _TBUF)
            def _():
                for c in store_descs(n, tb):
                    c.wait()

            posv = [[pos_v[pl.ds(sv * (_N * _D) + n * _D + db * _LANES,
                                 _LANES)]
                     for db in range(_D // _LANES)] for sv in range(_S)]
            tb_spl = jnp.full((_LANES,), tb, jnp.int32)

            @plsc.parallel_loop(0, _BAND // _S, unroll=2)
            def row4(q):
                r0 = q * _S
                for sv in range(_S):
                    r = par * _BAND + r0 + sv
                    r_spl = jnp.full((_LANES,), r0 + sv, jnp.int32)
                    for db in range(_D // _LANES):
                        vals = (stage_v[r, pl.ds(db * _LANES, _LANES)]
                                + posv[sv][db])
                        plsc.store_scatter(
                            tile_v, [tb_spl, d_vecs[db], r_spl], vals)

            for c in store_descs(n, tb):
                c.start()

            @pl.when(n + _GBUF - 1 < _N)
            def _():
                gather_desc(n + _GBUF - 1, (par + _GBUF - 1) % _GBUF).start()
        return carry

    lax.fori_loop(0, _N // _GBUF, step, 0)

    for c in store_descs(_N - 2, (_N - 2) % _TBUF):
        c.wait()
    for c in store_descs(_N - 1, (_N - 1) % _TBUF):
        c.wait()


_sc_call = functools.partial(
    pl.kernel,
    out_type=jax.ShapeDtypeStruct((_N, _DT, _NUM_WORKERS, 8, _BAND),
                                  jnp.float32),
    mesh=plsc.VectorSubcoreMesh(core_axis_name="core",
                                subcore_axis_name="subcore"),
    scratch_types=[
        pltpu.VMEM((_N, _BAND), jnp.int32),
        pltpu.VMEM((_S * _N * _D,), jnp.float32),
        pltpu.VMEM((_GBUF * _BAND, _D), jnp.float32),
        pltpu.VMEM((_TBUF, _D, _BAND + 1), jnp.float32),
        pltpu.SemaphoreType.DMA((_GBUF,)),
        pltpu.SemaphoreType.DMA((_TBUF,)),
    ],
    compiler_params=pltpu.CompilerParams(use_tc_tiling_on_sc=False,
                                         needs_layout_passes=False),
)(_sc_body)


def kernel(x, token_emb, pos_emb):
    tok_flat = token_emb.reshape(_S * _V, _D)
    offs = jnp.arange(_S, dtype=jnp.int32) * _V
    idx_t = (x.T.astype(jnp.int32)[:, :, None]
             + offs[None, None, :]).reshape(_N, _R)
    pos_flat = pos_emb[:, :_N, :].reshape(_S * _N * _D)
    out_t = _sc_call(idx_t, tok_flat, pos_flat)
    return out_t.transpose(2, 4, 0, 1, 3).reshape(_R, _N, _D)


# speedup vs baseline: 2.4169x; 1.2146x over previous
import functools

import jax
import jax.numpy as jnp
from jax import lax
from jax.experimental import pallas as pl
from jax.experimental.pallas import tpu as pltpu
from jax.experimental.pallas import tpu_sc as plsc

_B, _N = 1024, 200
_S, _V, _D = 4, 100000, 64
_LANES = 16
_R = _B * _S

_NUM_WORKERS = 32
_BAND = _R // _NUM_WORKERS
_GBUF = 4
_TBUF = 2
_DT = _D // 8
_TILE = _D * _BAND


def _sc_body(idx_hbm, tok_hbm, pos_hbm, out_hbm,
             idx_v, pos_v, stage_v, tile_v, gsem, ssem):
    wid = lax.axis_index("subcore") * 2 + lax.axis_index("core")
    col0 = pl.multiple_of(wid * _BAND, 8)

    pltpu.sync_copy(idx_hbm.at[:, pl.ds(col0, _BAND)], idx_v)
    pltpu.sync_copy(pos_hbm, pos_v)

    iota = lax.iota(jnp.int32, _LANES)
    d_vecs = [iota + db * _LANES for db in range(_D // _LANES)]

    def gather_desc(n, slot):
        return pltpu.make_async_copy(
            tok_hbm.at[idx_v.at[n]],
            stage_v.at[pl.ds(slot * _BAND, _BAND)], gsem.at[slot])

    def store_descs(n, tb):
        return [pltpu.make_async_copy(
            tile_v.at[tb, pl.ds(dt * 8, 8), pl.ds(0, _BAND)],
            out_hbm.at[n, dt, wid], ssem.at[tb]) for dt in range(_DT)]

    for n in range(_GBUF - 1):
        gather_desc(n, n).start()

    def step(n2, carry):
        for par in range(_GBUF):
            n = n2 * _GBUF + par
            tb = par % _TBUF
            gather_desc(n, par).wait()

            @pl.when(n >= _TBUF)
            def _():
                for c in store_descs(n, tb):
                    c.wait()

            posv = [[pos_v[pl.ds(sv * (_N * _D) + n * _D + db * _LANES,
                                 _LANES)]
                     for db in range(_D // _LANES)] for sv in range(_S)]
            tb_spl = jnp.full((_LANES,), tb, jnp.int32)

            @plsc.parallel_loop(0, _BAND // _S, unroll=1)
            def row4(q):
                r0 = q * _S
                for sv in range(_S):
                    r = par * _BAND + r0 + sv
                    r_spl = jnp.full((_LANES,), r0 + sv, jnp.int32)
                    for db in range(_D // _LANES):
                        vals = (stage_v[r, pl.ds(db * _LANES, _LANES)]
                                + posv[sv][db])
                        plsc.store_scatter(
                            tile_v, [tb_spl, d_vecs[db], r_spl], vals)

            for c in store_descs(n, tb):
                c.start()

            @pl.when(n + _GBUF - 1 < _N)
            def _():
                gather_desc(n + _GBUF - 1, (par + _GBUF - 1) % _GBUF).start()
        return carry

    lax.fori_loop(0, _N // _GBUF, step, 0)

    for c in store_descs(_N - 2, (_N - 2) % _TBUF):
        c.wait()
    for c in store_descs(_N - 1, (_N - 1) % _TBUF):
        c.wait()


_sc_call = functools.partial(
    pl.kernel,
    out_type=jax.ShapeDtypeStruct((_N, _DT, _NUM_WORKERS, 8, _BAND),
                                  jnp.float32),
    mesh=plsc.VectorSubcoreMesh(core_axis_name="core",
                                subcore_axis_name="subcore"),
    scratch_types=[
        pltpu.VMEM((_N, _BAND), jnp.int32),
        pltpu.VMEM((_S * _N * _D,), jnp.float32),
        pltpu.VMEM((_GBUF * _BAND, _D), jnp.float32),
        pltpu.VMEM((_TBUF, _D, _BAND + 1), jnp.float32),
        pltpu.SemaphoreType.DMA((_GBUF,)),
        pltpu.SemaphoreType.DMA((_TBUF,)),
    ],
    compiler_params=pltpu.CompilerParams(use_tc_tiling_on_sc=False,
                                         needs_layout_passes=False),
)(_sc_body)


def kernel(x, token_emb, pos_emb):
    tok_flat = token_emb.reshape(_S * _V, _D)
    offs = jnp.arange(_S, dtype=jnp.int32) * _V
    idx_t = (x.T.astype(jnp.int32)[:, :, None]
             + offs[None, None, :]).reshape(_N, _R)
    pos_flat = pos_emb[:, :_N, :].reshape(_S * _N * _D)
    out_t = _sc_call(idx_t, tok_flat, pos_flat)
    return out_t.transpose(2, 4, 0, 1, 3).reshape(_R, _N, _D)


# speedup vs baseline: 2.5212x; 1.0432x over previous
import functools

import jax
import jax.numpy as jnp
from jax import lax
from jax.experimental import pallas as pl
from jax.experimental.pallas import tpu as pltpu
from jax.experimental.pallas import tpu_sc as plsc

_B, _N = 1024, 200
_S, _V, _D = 4, 100000, 64
_LANES = 16
_R = _B * _S

_NUM_WORKERS = 32
_BAND = _R // _NUM_WORKERS
_GBUF = 4
_TBUF = 2
_DT = _D // 8
_TILE = _D * _BAND


def _sc_body(idx_hbm, tok_hbm, pos_hbm, out_hbm,
             idx_v, pos_v, stage_v, tile_v, gsem, ssem):
    wid = lax.axis_index("subcore") * 2 + lax.axis_index("core")
    col0 = pl.multiple_of(wid * _BAND, 8)

    pltpu.sync_copy(idx_hbm.at[:, pl.ds(col0, _BAND)], idx_v)
    pltpu.sync_copy(pos_hbm, pos_v)

    iota = lax.iota(jnp.int32, _LANES)
    d_vecs = [iota + db * _LANES for db in range(_D // _LANES)]

    def gather_desc(n, slot):
        return pltpu.make_async_copy(
            tok_hbm.at[idx_v.at[n]],
            stage_v.at[pl.ds(slot * _BAND, _BAND)], gsem.at[slot])

    def store_descs(n, tb):
        return [pltpu.make_async_copy(
            tile_v.at[tb, pl.ds(dt * 8, 8), pl.ds(0, _BAND)],
            out_hbm.at[n, dt, wid], ssem.at[tb]) for dt in range(_DT)]

    for n in range(_GBUF - 1):
        gather_desc(n, n).start()

    def step(n, carry):
        par = lax.rem(n, _GBUF)
        tb = lax.rem(n, _TBUF)
        gather_desc(n, par).wait()

        @pl.when(n >= _TBUF)
        def _():
            for c in store_descs(n, tb):
                c.wait()

        posv = [[pos_v[pl.ds(sv * (_N * _D) + n * _D + db * _LANES,
                             _LANES)]
                 for db in range(_D // _LANES)] for sv in range(_S)]
        tb_spl = jnp.full((_LANES,), tb, jnp.int32)
        base = par * _BAND

        @plsc.parallel_loop(0, _BAND // _S, unroll=1)
        def row4(q):
            r0 = q * _S
            for sv in range(_S):
                r = base + r0 + sv
                r_spl = jnp.full((_LANES,), r0 + sv, jnp.int32)
                for db in range(_D // _LANES):
                    vals = (stage_v[r, pl.ds(db * _LANES, _LANES)]
                            + posv[sv][db])
                    plsc.store_scatter(
                        tile_v, [tb_spl, d_vecs[db], r_spl], vals)

        for c in store_descs(n, tb):
            c.start()

        @pl.when(n + _GBUF - 1 < _N)
        def _():
            gather_desc(n + _GBUF - 1, lax.rem(n + _GBUF - 1, _GBUF)).start()
        return carry

    lax.fori_loop(0, _N, step, 0)

    for c in store_descs(_N - 2, (_N - 2) % _TBUF):
        c.wait()
    for c in store_descs(_N - 1, (_N - 1) % _TBUF):
        c.wait()


_sc_call = functools.partial(
    pl.kernel,
    out_type=jax.ShapeDtypeStruct((_N, _DT, _NUM_WORKERS, 8, _BAND),
                                  jnp.float32),
    mesh=plsc.VectorSubcoreMesh(core_axis_name="core",
                                subcore_axis_name="subcore"),
    scratch_types=[
        pltpu.VMEM((_N, _BAND), jnp.int32),
        pltpu.VMEM((_S * _N * _D,), jnp.float32),
        pltpu.VMEM((_GBUF * _BAND, _D), jnp.float32),
        pltpu.VMEM((_TBUF, _D, _BAND + 1), jnp.float32),
        pltpu.SemaphoreType.DMA((_GBUF,)),
        pltpu.SemaphoreType.DMA((_TBUF,)),
    ],
    compiler_params=pltpu.CompilerParams(use_tc_tiling_on_sc=False,
                                         needs_layout_passes=False),
)(_sc_body)


def kernel(x, token_emb, pos_emb):
    tok_flat = token_emb.reshape(_S * _V, _D)
    offs = jnp.arange(_S, dtype=jnp.int32) * _V
    idx_t = (x.T.astype(jnp.int32)[:, :, None]
             + offs[None, None, :]).reshape(_N, _R)
    pos_flat = pos_emb[:, :_N, :].reshape(_S * _N * _D)
    out_t = _sc_call(idx_t, tok_flat, pos_flat)
    return out_t.transpose(2, 4, 0, 1, 3).reshape(_R, _N, _D)
